# Initial kernel scaffold; baseline (speedup 1.0000x reference)
import jax
import jax.numpy as jnp
from jax.experimental import pallas as pl


def kernel(x, edge_index, W1, b1, Wg, a_src, a_dst, bg, Wmu, bmu, Wls, bls):
    raise NotImplementedError("write your pallas kernel here")


# speedup vs baseline: 32.0866x; 32.0866x over previous
import functools

import jax
import jax.numpy as jnp
from jax import lax
from jax.experimental import pallas as pl
from jax.experimental.pallas import tpu as pltpu
from jax.experimental.pallas import tpu_sc as plsc

N = 10000
E = 320000
DIN = 128
HID = 128
HEADS = 4
DH = HID // HEADS
DOUT = 64

NC = 2
NS = 16
EPC = E // NC
EPT = EPC // NS
CH = 80
NCHUNK = EPT // CH
RPT = 624
TAIL0 = RPT * NS
TAILN = N - TAIL0

ROWB = 2000
GRID = N // ROWB

_mesh = plsc.VectorSubcoreMesh(core_axis_name="c", subcore_axis_name="s")
_f32 = jnp.float32


def _lane_perm(vec, idxvec):
    dnums = lax.GatherDimensionNumbers(
        offset_dims=(), collapsed_slice_dims=(0,), start_index_map=(0,))
    return lax.gather(vec, idxvec.reshape(16, 1), dnums, slice_sizes=(1,),
                      mode=lax.GatherScatterMode.PROMISE_IN_BOUNDS)


def _zero_rows(buf, ncol_vecs):
    z16 = jnp.zeros((16,), _f32)

    @pl.loop(0, CH)
    def _(r):
        for j in range(ncol_vecs):
            buf[r, pl.ds(16 * j, 16)] = z16


def _fill_range(src_buf, acc, row0, s):
    nfull = RPT // CH
    rem = RPT - nfull * CH
    for b in range(nfull):
        pltpu.sync_copy(src_buf, acc.at[pl.ds(row0 + b * CH, CH)])
    if rem:
        pltpu.sync_copy(src_buf.at[pl.ds(0, rem)],
                        acc.at[pl.ds(row0 + nfull * CH, rem)])

    @pl.when(s == NS - 1)
    def _():
        pltpu.sync_copy(src_buf.at[pl.ds(0, TAILN)],
                        acc.at[pl.ds(TAIL0, TAILN)])


def _dump_range(acc, out_hbm, c, s, row0):
    pltpu.sync_copy(acc.at[pl.ds(row0, RPT)],
                    out_hbm.at[c, pl.ds(row0, RPT)])

    @pl.when(s == NS - 1)
    def _():
        pltpu.sync_copy(acc.at[pl.ds(TAIL0, TAILN)],
                        out_hbm.at[c, pl.ds(TAIL0, TAILN)])


def _build_deg():
    @functools.partial(
        pl.kernel,
        out_type=jax.ShapeDtypeStruct((NC, N, 16), _f32),
        mesh=_mesh,
        scratch_types=[
            pltpu.VMEM((CH,), jnp.int32),
            pltpu.VMEM((CH, 16), _f32),
            pltpu.MemorySpace.VMEM_SHARED((N, 16), _f32),
        ],
    )
    def deg(d_hbm, out_hbm, didx, obuf, acc):
        c = lax.axis_index("c")
        s = lax.axis_index("s")
        row0 = s * RPT
        _zero_rows(obuf, 1)
        _fill_range(obuf, acc, row0, s)
        onerow = jnp.where(lax.iota(jnp.int32, 16) == 0,
                           jnp.float32(1.0), jnp.float32(0.0))

        @pl.loop(0, CH)
        def _(r):
            obuf[r] = onerow

        plsc.subcore_barrier()

        @pl.loop(0, NCHUNK)
        def _(i):
            base = c * EPC + s * EPT + i * CH
            pltpu.sync_copy(d_hbm.at[pl.ds(base, CH)], didx)
            pltpu.sync_copy(obuf, acc.at[didx], add=True)

        plsc.subcore_barrier()
        _dump_range(acc, out_hbm, c, s, row0)

    return deg


def _build_gcn():
    @functools.partial(
        pl.kernel,
        out_type=jax.ShapeDtypeStruct((NC, N, HID), _f32),
        mesh=_mesh,
        scratch_types=[
            pltpu.VMEM((CH,), jnp.int32),
            pltpu.VMEM((CH,), jnp.int32),
            pltpu.VMEM((CH, HID), _f32),
            pltpu.MemorySpace.VMEM_SHARED((N, HID), _f32),
            pltpu.SemaphoreType.DMA,
        ],
    )
    def gcn(tab_hbm, s_hbm, d_hbm, out_hbm, sidx, didx, rows, acc, sem):
        c = lax.axis_index("c")
        s = lax.axis_index("s")
        row0 = s * RPT
        _zero_rows(rows, HID // 16)
        _fill_range(rows, acc, row0, s)
        plsc.subcore_barrier()

        @pl.loop(0, NCHUNK)
        def _(i):
            base = c * EPC + s * EPT + i * CH
            pltpu.sync_copy(s_hbm.at[pl.ds(base, CH)], sidx)
            pltpu.sync_copy(d_hbm.at[pl.ds(base, CH)], didx)
            pltpu.async_copy(tab_hbm.at[sidx], rows, sem).wait()
            pltpu.sync_copy(rows, acc.at[didx], add=True)

        plsc.subcore_barrier()
        _dump_range(acc, out_hbm, c, s, row0)

    return gcn


def _build_den():
    @functools.partial(
        pl.kernel,
        out_type=[
            jax.ShapeDtypeStruct((NC, N, 16), _f32),
            jax.ShapeDtypeStruct((E, 16), _f32),
        ],
        mesh=_mesh,
        scratch_types=[
            pltpu.VMEM((CH,), jnp.int32),
            pltpu.VMEM((CH,), jnp.int32),
            pltpu.VMEM((CH, HID), _f32),
            pltpu.VMEM((CH, HID), _f32),
            pltpu.VMEM((CH, 16), _f32),
            pltpu.VMEM((128,), _f32),
            pltpu.MemorySpace.VMEM_SHARED((N, 16), _f32),
            pltpu.SemaphoreType.DMA,
        ],
    )
    def den(aux_hbm, m_hbm, s_hbm, d_hbm,
            den_hbm, ex_hbm,
            sidx, didx, auxs, auxd, exv, mv, acc, sem):
        c = lax.axis_index("c")
        s = lax.axis_index("s")
        row0 = s * RPT
        _zero_rows(exv, 1)
        _fill_range(exv, acc, row0, s)
        pltpu.sync_copy(m_hbm, mv)
        plsc.subcore_barrier()
        mreg = mv[pl.ds(0, 16)]
        lane = lax.iota(jnp.int32, 16)
        perm = jnp.where(lane < 4, lane + 4, 8)

        @pl.loop(0, NCHUNK)
        def _(i):
            base = c * EPC + s * EPT + i * CH
            pltpu.sync_copy(s_hbm.at[pl.ds(base, CH)], sidx)
            pltpu.sync_copy(d_hbm.at[pl.ds(base, CH)], didx)
            ca = pltpu.async_copy(aux_hbm.at[sidx], auxs, sem)
            cb = pltpu.async_copy(aux_hbm.at[didx], auxd, sem)
            ca.wait()
            cb.wait()

            @pl.loop(0, CH)
            def _(k):
                e = auxs[k, pl.ds(0, 16)] + _lane_perm(auxd[k, pl.ds(0, 16)],
                                                       perm)
                e = jnp.maximum(e, 0.2 * e)
                exv[k] = jnp.exp(e - mreg)

            pltpu.sync_copy(exv, acc.at[didx], add=True)
            pltpu.sync_copy(exv, ex_hbm.at[pl.ds(base, CH)])

        plsc.subcore_barrier()
        _dump_range(acc, den_hbm, c, s, row0)

    return den


def _build_gatw():
    @functools.partial(
        pl.kernel,
        out_type=jax.ShapeDtypeStruct((NC, N, HID), _f32),
        mesh=_mesh,
        scratch_types=[
            pltpu.VMEM((CH,), jnp.int32),
            pltpu.VMEM((CH,), jnp.int32),
            pltpu.VMEM((CH, HID), _f32),
            pltpu.VMEM((CH, 16), _f32),
            pltpu.MemorySpace.VMEM_SHARED((N, HID), _f32),
            pltpu.SemaphoreType.DMA,
        ],
    )
    def gatw(g_hbm, ex_hbm, s_hbm, d_hbm, out_hbm,
             sidx, didx, rows, exv, acc, sem):
        c = lax.axis_index("c")
        s = lax.axis_index("s")
        row0 = s * RPT
        _zero_rows(rows, HID // 16)
        _fill_range(rows, acc, row0, s)
        plsc.subcore_barrier()
        lane = lax.iota(jnp.int32, 16)
        zero = lane * 0

        @pl.loop(0, NCHUNK)
        def _(i):
            base = c * EPC + s * EPT + i * CH
            pltpu.sync_copy(s_hbm.at[pl.ds(base, CH)], sidx)
            pltpu.sync_copy(d_hbm.at[pl.ds(base, CH)], didx)
            cg = pltpu.async_copy(g_hbm.at[sidx], rows, sem)
            pltpu.sync_copy(ex_hbm.at[pl.ds(base, CH)], exv)
            cg.wait()

            @pl.loop(0, CH)
            def _(k):
                ex = exv[k]
                for j in range(HID // 16):
                    al = _lane_perm(ex, zero + (j // 2))
                    rows[k, pl.ds(16 * j, 16)] = rows[k, pl.ds(16 * j, 16)] * al

            pltpu.sync_copy(rows, acc.at[didx], add=True)

        plsc.subcore_barrier()
        _dump_range(acc, out_hbm, c, s, row0)

    return gatw


_deg_call = _build_deg()
_gcn_call = _build_gcn()
_den_call = _build_den()
_gatw_call = _build_gatw()


def _rowspec():
    return pl.BlockSpec((ROWB, 128), lambda i: (i, 0))


def _wspec():
    return pl.BlockSpec((128, 128), lambda i: (0, 0))


def _bspec():
    return pl.BlockSpec((1, 128), lambda i: (0, 0))


def _prep1(x, disb, W1):
    def body(x_ref, db_ref, w_ref, o_ref):
        o_ref[...] = jnp.dot(db_ref[...] * x_ref[...], w_ref[...],
                             preferred_element_type=_f32)

    return pl.pallas_call(
        body,
        grid=(GRID,),
        in_specs=[_rowspec(), _rowspec(), _wspec()],
        out_specs=_rowspec(),
        out_shape=jax.ShapeDtypeStruct((N, 128), _f32),
    )(x, disb, W1)


def _mid(p0, p1, h1p, disb, b1, Wg, P):
    def body(p0_ref, p1_ref, h_ref, db_ref, b1_ref, wg_ref, p_ref,
             g_o, aux_o, m_o):
        a = p0_ref[...] + p1_ref[...] + h_ref[...]
        z = db_ref[...] * a + b1_ref[...]
        h = 0.5 * z * (1.0 + lax.erf(z * jnp.float32(0.7071067811865476)))
        g = jnp.dot(h, wg_ref[...], preferred_element_type=_f32)
        g_o[...] = g
        aux = jnp.dot(g, p_ref[...], preferred_element_type=_f32)
        aux_o[...] = aux
        cur = jnp.broadcast_to(jnp.max(aux, axis=0, keepdims=True), (8, 128))
        i = pl.program_id(0)

        @pl.when(i == 0)
        def _():
            m_o[...] = cur

        @pl.when(i > 0)
        def _():
            m_o[...] = jnp.maximum(m_o[...], cur)

    return pl.pallas_call(
        body,
        grid=(GRID,),
        in_specs=[_rowspec(), _rowspec(), _rowspec(), _rowspec(), _bspec(),
                  _wspec(), _wspec()],
        out_specs=[_rowspec(), _rowspec(),
                   pl.BlockSpec((8, 128), lambda i: (0, 0))],
        out_shape=[jax.ShapeDtypeStruct((N, 128), _f32),
                   jax.ShapeDtypeStruct((N, 128), _f32),
                   jax.ShapeDtypeStruct((8, 128), _f32)],
    )(p0, p1, h1p, disb, b1, Wg, P)


def _finprep(o0, o1, exsrep, g, denrep, bg, Wcat, disb):
    def body(o0_ref, o1_ref, ex_ref, g_ref, dn_ref, bg_ref, w_ref, db_ref,
             out_ref):
        num = o0_ref[...] + o1_ref[...] + ex_ref[...] * g_ref[...]
        h2 = jnp.maximum(num / dn_ref[...] + bg_ref[...], 0.0)
        out_ref[...] = jnp.dot(h2, w_ref[...],
                               preferred_element_type=_f32) * db_ref[...]

    return pl.pallas_call(
        body,
        grid=(GRID,),
        in_specs=[_rowspec()] * 5 + [_bspec(), _wspec(), _rowspec()],
        out_specs=_rowspec(),
        out_shape=jax.ShapeDtypeStruct((N, 128), _f32),
    )(o0, o1, exsrep, g, denrep, bg, Wcat, disb)


def _fin(r0, r1, hcp, disb, bcat):
    def body(r0_ref, r1_ref, h_ref, db_ref, b_ref, out_ref):
        out_ref[...] = (db_ref[...] * (r0_ref[...] + r1_ref[...] + h_ref[...])
                        + b_ref[...])

    return pl.pallas_call(
        body,
        grid=(GRID,),
        in_specs=[_rowspec()] * 4 + [_bspec()],
        out_specs=_rowspec(),
        out_shape=jax.ShapeDtypeStruct((N, 128), _f32),
    )(r0, r1, hcp, disb, bcat)


def kernel(x, edge_index, W1, b1, Wg, a_src, a_dst, bg, Wmu, bmu, Wls, bls):
    s = edge_index[0]
    d = edge_index[1]

    degp = _deg_call(d)
    deg = degp[0, :, 0] + degp[1, :, 0] + 1.0
    dis = lax.rsqrt(deg)
    disb = jnp.broadcast_to(dis[:, None], (N, 128))

    h1p = _prep1(x, disb, W1)
    p = _gcn_call(h1p, s, d)

    eye = jnp.eye(HEADS, 128, dtype=_f32)
    Asp = (a_src[:, :, None] * eye[:, None, :]).reshape(HID, 128)
    Adp = (a_dst[:, :, None] * eye[:, None, :]).reshape(HID, 128)
    P = Asp + jnp.roll(Adp, HEADS, axis=1)

    g, aux, mout = _mid(p[0], p[1], h1p, disb, b1[None, :], Wg, P)

    m4 = jnp.maximum(mout[0, :HEADS] + mout[0, HEADS:2 * HEADS], 0.0)
    mvec = jnp.concatenate([m4, jnp.full((124,), 1e30, _f32)])

    asn = aux[:, :HEADS]
    adn = aux[:, HEADS:2 * HEADS]
    es = asn + adn
    es = jnp.maximum(es, 0.2 * es)
    exs = jnp.exp(es - m4[None, :])
    exsrep = jnp.repeat(exs, DH, axis=1)

    denp, exe = _den_call(aux, mvec, s, d)
    outp = _gatw_call(g, exe, s, d)

    den = denp[0, :, :HEADS] + denp[1, :, :HEADS] + exs
    denrep = jnp.repeat(den, DH, axis=1)

    Wcat = jnp.concatenate([Wmu, Wls], axis=1)
    hcp = _finprep(outp[0], outp[1], exsrep, g, denrep, bg[None, :],
                   Wcat, disb)

    r = _gcn_call(hcp, s, d)
    bcat = jnp.concatenate([bmu, bls])[None, :]
    out = _fin(r[0], r[1], hcp, disb, bcat)
    return out[:, :DOUT], out[:, DOUT:]


# speedup vs baseline: 37.8504x; 1.1796x over previous
import functools

import jax
import jax.numpy as jnp
from jax import lax
from jax.experimental import pallas as pl
from jax.experimental.pallas import tpu as pltpu
from jax.experimental.pallas import tpu_sc as plsc

N = 10000
E = 320000
DIN = 128
HID = 128
HEADS = 4
DH = HID // HEADS
DOUT = 64

NC = 2
NS = 16
EPC = E // NC
EPT = EPC // NS
CH = 80
NCHUNK = EPT // CH
RPT = 624
TAIL0 = RPT * NS
TAILN = N - TAIL0

ROWB = 2000
GRID = N // ROWB

_mesh = plsc.VectorSubcoreMesh(core_axis_name="c", subcore_axis_name="s")
_f32 = jnp.float32


def _lane_perm(vec, idxvec):
    dnums = lax.GatherDimensionNumbers(
        offset_dims=(), collapsed_slice_dims=(0,), start_index_map=(0,))
    return lax.gather(vec, idxvec.reshape(16, 1), dnums, slice_sizes=(1,),
                      mode=lax.GatherScatterMode.PROMISE_IN_BOUNDS)


def _zero_rows(buf, ncol_vecs):
    z16 = jnp.zeros((16,), _f32)

    @pl.loop(0, CH)
    def _(r):
        for j in range(ncol_vecs):
            buf[r, pl.ds(16 * j, 16)] = z16


def _fill_range(src_buf, acc, row0, s):
    nfull = RPT // CH
    rem = RPT - nfull * CH
    for b in range(nfull):
        pltpu.sync_copy(src_buf, acc.at[pl.ds(row0 + b * CH, CH)])
    if rem:
        pltpu.sync_copy(src_buf.at[pl.ds(0, rem)],
                        acc.at[pl.ds(row0 + nfull * CH, rem)])

    @pl.when(s == NS - 1)
    def _():
        pltpu.sync_copy(src_buf.at[pl.ds(0, TAILN)],
                        acc.at[pl.ds(TAIL0, TAILN)])


def _dump_range(acc, out_hbm, c, s, row0):
    pltpu.sync_copy(acc.at[pl.ds(row0, RPT)],
                    out_hbm.at[c, pl.ds(row0, RPT)])

    @pl.when(s == NS - 1)
    def _():
        pltpu.sync_copy(acc.at[pl.ds(TAIL0, TAILN)],
                        out_hbm.at[c, pl.ds(TAIL0, TAILN)])


def _build_deg():
    @functools.partial(
        pl.kernel,
        out_type=jax.ShapeDtypeStruct((NC, N, 16), _f32),
        mesh=_mesh,
        scratch_types=[
            pltpu.VMEM((CH,), jnp.int32),
            pltpu.VMEM((CH, 16), _f32),
            pltpu.MemorySpace.VMEM_SHARED((N, 16), _f32),
        ],
    )
    def deg(d_hbm, out_hbm, didx, obuf, acc):
        c = lax.axis_index("c")
        s = lax.axis_index("s")
        row0 = s * RPT
        _zero_rows(obuf, 1)
        _fill_range(obuf, acc, row0, s)
        onerow = jnp.where(lax.iota(jnp.int32, 16) == 0,
                           jnp.float32(1.0), jnp.float32(0.0))

        @pl.loop(0, CH)
        def _(r):
            obuf[r] = onerow

        plsc.subcore_barrier()

        @pl.loop(0, NCHUNK)
        def _(i):
            base = c * EPC + s * EPT + i * CH
            pltpu.sync_copy(d_hbm.at[pl.ds(base, CH)], didx)
            pltpu.sync_copy(obuf, acc.at[didx], add=True)

        plsc.subcore_barrier()
        _dump_range(acc, out_hbm, c, s, row0)

    return deg


def _build_gcn():
    @functools.partial(
        pl.kernel,
        out_type=jax.ShapeDtypeStruct((NC, N, HID), _f32),
        mesh=_mesh,
        scratch_types=[
            pltpu.VMEM((CH,), jnp.int32),
            pltpu.VMEM((CH,), jnp.int32),
            pltpu.VMEM((CH, HID), _f32),
            pltpu.VMEM((CH,), jnp.int32),
            pltpu.VMEM((CH,), jnp.int32),
            pltpu.VMEM((CH, HID), _f32),
            pltpu.MemorySpace.VMEM_SHARED((N, HID), _f32),
            pltpu.SemaphoreType.DMA,
            pltpu.SemaphoreType.DMA,
        ],
    )
    def gcn(tab_hbm, s_hbm, d_hbm, out_hbm,
            sidx0, didx0, rows0, sidx1, didx1, rows1, acc, sem0, sem1):
        c = lax.axis_index("c")
        s = lax.axis_index("s")
        row0 = s * RPT
        _zero_rows(rows0, HID // 16)
        _fill_range(rows0, acc, row0, s)
        plsc.subcore_barrier()
        sidx = (sidx0, sidx1)
        didx = (didx0, didx1)
        rows = (rows0, rows1)
        sem = (sem0, sem1)
        ebase = c * EPC + s * EPT

        def stage(b, i):
            base = ebase + i * CH
            pltpu.sync_copy(s_hbm.at[pl.ds(base, CH)], sidx[b])
            pltpu.sync_copy(d_hbm.at[pl.ds(base, CH)], didx[b])
            pltpu.async_copy(tab_hbm.at[sidx[b]], rows[b], sem[b])

        def finish(b):
            pltpu.make_async_copy(tab_hbm.at[sidx[b]], rows[b],
                                  sem[b]).wait()
            pltpu.sync_copy(rows[b], acc.at[didx[b]], add=True)

        stage(0, 0)

        @pl.loop(0, NCHUNK - 1, step=2)
        def _(i):
            stage(1, i + 1)
            finish(0)
            stage(0, i + 2)
            finish(1)

        finish(0)
        plsc.subcore_barrier()
        _dump_range(acc, out_hbm, c, s, row0)

    return gcn


def _build_den():
    @functools.partial(
        pl.kernel,
        out_type=[
            jax.ShapeDtypeStruct((NC, N, 16), _f32),
            jax.ShapeDtypeStruct((E, 16), _f32),
        ],
        mesh=_mesh,
        scratch_types=[
            pltpu.VMEM((CH,), jnp.int32),
            pltpu.VMEM((CH,), jnp.int32),
            pltpu.VMEM((CH, HID), _f32),
            pltpu.VMEM((CH, HID), _f32),
            pltpu.VMEM((CH, 16), _f32),
            pltpu.VMEM((128,), _f32),
            pltpu.MemorySpace.VMEM_SHARED((N, 16), _f32),
            pltpu.SemaphoreType.DMA,
        ],
    )
    def den(aux_hbm, m_hbm, s_hbm, d_hbm,
            den_hbm, ex_hbm,
            sidx, didx, auxs, auxd, exv, mv, acc, sem):
        c = lax.axis_index("c")
        s = lax.axis_index("s")
        row0 = s * RPT
        _zero_rows(exv, 1)
        _fill_range(exv, acc, row0, s)
        pltpu.sync_copy(m_hbm, mv)
        plsc.subcore_barrier()
        mreg = mv[pl.ds(0, 16)]
        lane = lax.iota(jnp.int32, 16)
        perm = jnp.where(lane < 4, lane + 4, 8)

        @pl.loop(0, NCHUNK)
        def _(i):
            base = c * EPC + s * EPT + i * CH
            pltpu.sync_copy(s_hbm.at[pl.ds(base, CH)], sidx)
            pltpu.sync_copy(d_hbm.at[pl.ds(base, CH)], didx)
            ca = pltpu.async_copy(aux_hbm.at[sidx], auxs, sem)
            cb = pltpu.async_copy(aux_hbm.at[didx], auxd, sem)
            ca.wait()
            cb.wait()

            @pl.loop(0, CH)
            def _(k):
                e = auxs[k, pl.ds(0, 16)] + _lane_perm(auxd[k, pl.ds(0, 16)],
                                                       perm)
                e = jnp.maximum(e, 0.2 * e)
                exv[k] = jnp.exp(e - mreg)

            pltpu.sync_copy(exv, acc.at[didx], add=True)
            pltpu.sync_copy(exv, ex_hbm.at[pl.ds(base, CH)])

        plsc.subcore_barrier()
        _dump_range(acc, den_hbm, c, s, row0)

    return den


def _build_gatw():
    @functools.partial(
        pl.kernel,
        out_type=jax.ShapeDtypeStruct((NC, N, HID), _f32),
        mesh=_mesh,
        scratch_types=[
            pltpu.VMEM((CH,), jnp.int32),
            pltpu.VMEM((CH,), jnp.int32),
            pltpu.VMEM((CH, HID), _f32),
            pltpu.VMEM((CH, 16), _f32),
            pltpu.MemorySpace.VMEM_SHARED((N, HID), _f32),
            pltpu.SemaphoreType.DMA,
        ],
    )
    def gatw(g_hbm, ex_hbm, s_hbm, d_hbm, out_hbm,
             sidx, didx, rows, exv, acc, sem):
        c = lax.axis_index("c")
        s = lax.axis_index("s")
        row0 = s * RPT
        _zero_rows(rows, HID // 16)
        _fill_range(rows, acc, row0, s)
        plsc.subcore_barrier()
        lane = lax.iota(jnp.int32, 16)
        zero = lane * 0

        @pl.loop(0, NCHUNK)
        def _(i):
            base = c * EPC + s * EPT + i * CH
            pltpu.sync_copy(s_hbm.at[pl.ds(base, CH)], sidx)
            pltpu.sync_copy(d_hbm.at[pl.ds(base, CH)], didx)
            cg = pltpu.async_copy(g_hbm.at[sidx], rows, sem)
            pltpu.sync_copy(ex_hbm.at[pl.ds(base, CH)], exv)
            cg.wait()

            @pl.loop(0, CH)
            def _(k):
                ex = exv[k]
                for j in range(HID // 16):
                    al = _lane_perm(ex, zero + (j // 2))
                    rows[k, pl.ds(16 * j, 16)] = rows[k, pl.ds(16 * j, 16)] * al

            pltpu.sync_copy(rows, acc.at[didx], add=True)

        plsc.subcore_barrier()
        _dump_range(acc, out_hbm, c, s, row0)

    return gatw


_deg_call = _build_deg()
_gcn_call = _build_gcn()
_den_call = _build_den()
_gatw_call = _build_gatw()


def _rowspec():
    return pl.BlockSpec((ROWB, 128), lambda i: (i, 0))


def _wspec():
    return pl.BlockSpec((128, 128), lambda i: (0, 0))


def _bspec():
    return pl.BlockSpec((1, 128), lambda i: (0, 0))


def _prep1(x, disb, W1):
    def body(x_ref, db_ref, w_ref, o_ref):
        o_ref[...] = jnp.dot(db_ref[...] * x_ref[...], w_ref[...],
                             preferred_element_type=_f32)

    return pl.pallas_call(
        body,
        grid=(GRID,),
        in_specs=[_rowspec(), _rowspec(), _wspec()],
        out_specs=_rowspec(),
        out_shape=jax.ShapeDtypeStruct((N, 128), _f32),
    )(x, disb, W1)


def _mid(p0, p1, h1p, disb, b1, Wg, P):
    def body(p0_ref, p1_ref, h_ref, db_ref, b1_ref, wg_ref, p_ref,
             g_o, aux_o, m_o):
        a = p0_ref[...] + p1_ref[...] + h_ref[...]
        z = db_ref[...] * a + b1_ref[...]
        h = 0.5 * z * (1.0 + lax.erf(z * jnp.float32(0.7071067811865476)))
        g = jnp.dot(h, wg_ref[...], preferred_element_type=_f32)
        g_o[...] = g
        aux = jnp.dot(g, p_ref[...], preferred_element_type=_f32)
        aux_o[...] = aux
        cur = jnp.broadcast_to(jnp.max(aux, axis=0, keepdims=True), (8, 128))
        i = pl.program_id(0)

        @pl.when(i == 0)
        def _():
            m_o[...] = cur

        @pl.when(i > 0)
        def _():
            m_o[...] = jnp.maximum(m_o[...], cur)

    return pl.pallas_call(
        body,
        grid=(GRID,),
        in_specs=[_rowspec(), _rowspec(), _rowspec(), _rowspec(), _bspec(),
                  _wspec(), _wspec()],
        out_specs=[_rowspec(), _rowspec(),
                   pl.BlockSpec((8, 128), lambda i: (0, 0))],
        out_shape=[jax.ShapeDtypeStruct((N, 128), _f32),
                   jax.ShapeDtypeStruct((N, 128), _f32),
                   jax.ShapeDtypeStruct((8, 128), _f32)],
    )(p0, p1, h1p, disb, b1, Wg, P)


def _finprep(o0, o1, exsrep, g, denrep, bg, Wcat, disb):
    def body(o0_ref, o1_ref, ex_ref, g_ref, dn_ref, bg_ref, w_ref, db_ref,
             out_ref):
        num = o0_ref[...] + o1_ref[...] + ex_ref[...] * g_ref[...]
        h2 = jnp.maximum(num / dn_ref[...] + bg_ref[...], 0.0)
        out_ref[...] = jnp.dot(h2, w_ref[...],
                               preferred_element_type=_f32) * db_ref[...]

    return pl.pallas_call(
        body,
        grid=(GRID,),
        in_specs=[_rowspec()] * 5 + [_bspec(), _wspec(), _rowspec()],
        out_specs=_rowspec(),
        out_shape=jax.ShapeDtypeStruct((N, 128), _f32),
    )(o0, o1, exsrep, g, denrep, bg, Wcat, disb)


def _fin(r0, r1, hcp, disb, bcat):
    def body(r0_ref, r1_ref, h_ref, db_ref, b_ref, out_ref):
        out_ref[...] = (db_ref[...] * (r0_ref[...] + r1_ref[...] + h_ref[...])
                        + b_ref[...])

    return pl.pallas_call(
        body,
        grid=(GRID,),
        in_specs=[_rowspec()] * 4 + [_bspec()],
        out_specs=_rowspec(),
        out_shape=jax.ShapeDtypeStruct((N, 128), _f32),
    )(r0, r1, hcp, disb, bcat)


def kernel(x, edge_index, W1, b1, Wg, a_src, a_dst, bg, Wmu, bmu, Wls, bls):
    s = edge_index[0]
    d = edge_index[1]

    degp = _deg_call(d)
    deg = degp[0, :, 0] + degp[1, :, 0] + 1.0
    dis = lax.rsqrt(deg)
    disb = jnp.broadcast_to(dis[:, None], (N, 128))

    h1p = _prep1(x, disb, W1)
    p = _gcn_call(h1p, s, d)

    eye = jnp.eye(HEADS, 128, dtype=_f32)
    Asp = (a_src[:, :, None] * eye[:, None, :]).reshape(HID, 128)
    Adp = (a_dst[:, :, None] * eye[:, None, :]).reshape(HID, 128)
    P = Asp + jnp.roll(Adp, HEADS, axis=1)

    g, aux, mout = _mid(p[0], p[1], h1p, disb, b1[None, :], Wg, P)

    m4 = jnp.maximum(mout[0, :HEADS] + mout[0, HEADS:2 * HEADS], 0.0)
    mvec = jnp.concatenate([m4, jnp.full((124,), 1e30, _f32)])

    asn = aux[:, :HEADS]
    adn = aux[:, HEADS:2 * HEADS]
    es = asn + adn
    es = jnp.maximum(es, 0.2 * es)
    exs = jnp.exp(es - m4[None, :])
    exsrep = jnp.repeat(exs, DH, axis=1)

    denp, exe = _den_call(aux, mvec, s, d)
    outp = _gatw_call(g, exe, s, d)

    den = denp[0, :, :HEADS] + denp[1, :, :HEADS] + exs
    denrep = jnp.repeat(den, DH, axis=1)

    Wcat = jnp.concatenate([Wmu, Wls], axis=1)
    hcp = _finprep(outp[0], outp[1], exsrep, g, denrep, bg[None, :],
                   Wcat, disb)

    r = _gcn_call(hcp, s, d)
    bcat = jnp.concatenate([bmu, bls])[None, :]
    out = _fin(r[0], r[1], hcp, disb, bcat)
    return out[:, :DOUT], out[:, DOUT:]
